# Initial kernel scaffold; baseline (speedup 1.0000x reference)
import jax
import jax.numpy as jnp
from jax.experimental import pallas as pl


def kernel(x, rotation_matrix, codebooks):
    raise NotImplementedError("write your pallas kernel here")


# speedup vs baseline: 1.1130x; 1.1130x over previous
import jax
import jax.numpy as jnp
from jax import lax
from jax.experimental import pallas as pl
from jax.experimental.pallas import tpu as pltpu

_NUM_CB = 4
_K = 1024
_D = 64
_TB = 1024
_BETA = 0.25


def _vq_block(xt_ref, xraw_ref, rot_ref, cb_ref, w2_ref, q_ref, part_ref):
    xr = lax.dot_general(xt_ref[...], rot_ref[...], (((1,), (0,)), ((), ())),
                         preferred_element_type=jnp.float32)
    residual = xr
    qsum = jnp.zeros_like(xr)
    iota = lax.broadcasted_iota(jnp.int32, (_TB, _K), 1)
    idxs = []
    for i in range(_NUM_CB):
        cb = cb_ref[i]
        w2row = w2_ref[i]
        rw = lax.dot_general(residual, cb, (((1,), (1,)), ((), ())),
                             preferred_element_type=jnp.float32)
        rsq = jnp.sum(residual * residual, axis=1, keepdims=True)
        d = (rsq + w2row) - 2.0 * rw
        dmin = jnp.min(d, axis=1, keepdims=True)
        idx = jnp.min(jnp.where(d <= dmin, iota, _K), axis=1, keepdims=True)
        oh = (iota == idx).astype(jnp.float32)
        qi = lax.dot_general(oh, cb, (((1,), (0,)), ((), ())),
                             preferred_element_type=jnp.float32)
        residual = residual - qi
        qsum = qsum + qi
        idxs.append(idx)
    q_ref[...] = qsum

    ent = jnp.zeros((_TB, 1), jnp.float32)
    for j in range(_NUM_CB):
        m = jnp.zeros((_TB, 1), jnp.float32)
        for k in range(_NUM_CB):
            m = m + (idxs[j] == idxs[k]).astype(jnp.float32)
        ent = ent - 0.25 * jnp.log(m * 0.25 + 1e-10)
    ent_sum = jnp.sum(ent)

    diff = qsum - xraw_ref[...]
    sq_sum = jnp.sum(diff * diff)

    lane = lax.broadcasted_iota(jnp.int32, (1, 128), 1)
    row = jnp.where(lane == 0, sq_sum, jnp.where(lane == 1, ent_sum, 0.0))
    part_ref[...] = row.reshape(1, 1, 128)


def kernel(x, rotation_matrix, codebooks):
    B, C, H, W = x.shape
    N = B * H * W
    xt = jnp.transpose(x, (0, 2, 3, 1)).reshape(N, _D)
    xraw = x.reshape(N, _D)
    w2 = jnp.sum(codebooks * codebooks, axis=-1)[:, None, :]
    nblk = N // _TB
    q, part = pl.pallas_call(
        _vq_block,
        grid=(nblk,),
        in_specs=[
            pl.BlockSpec((_TB, _D), lambda i: (i, 0)),
            pl.BlockSpec((_TB, _D), lambda i: (i, 0)),
            pl.BlockSpec((_D, _D), lambda i: (0, 0)),
            pl.BlockSpec((_NUM_CB, _K, _D), lambda i: (0, 0, 0)),
            pl.BlockSpec((_NUM_CB, 1, _K), lambda i: (0, 0, 0)),
        ],
        out_specs=[
            pl.BlockSpec((_TB, _D), lambda i: (i, 0)),
            pl.BlockSpec((1, 1, 128), lambda i: (i, 0, 0)),
        ],
        out_shape=[
            jax.ShapeDtypeStruct((N, _D), jnp.float32),
            jax.ShapeDtypeStruct((nblk, 1, 128), jnp.float32),
        ],
    )(xt, xraw, rotation_matrix, codebooks, w2)
    sq_sum = jnp.sum(part[:, 0, 0])
    ent_sum = jnp.sum(part[:, 0, 1])
    loss = (1.0 + _BETA) * sq_sum / jnp.float32(x.size)
    perplexity = jnp.exp(ent_sum)
    quant_out = x + lax.stop_gradient(q.reshape(x.shape) - x)
    return loss, quant_out, perplexity


# speedup vs baseline: 1.1488x; 1.0322x over previous
import jax
import jax.numpy as jnp
from jax import lax
from jax.experimental import pallas as pl
from jax.experimental.pallas import tpu as pltpu

_NUM_CB = 4
_K = 1024
_D = 64
_TB = 1024
_BETA = 0.25


def _vq_block(xb_ref, xraw_ref, rot_ref, cb_ref, ncb_ref, w2_ref,
              out_ref, part_ref):
    xt = jnp.transpose(xb_ref[0], (1, 0))
    xr = lax.dot_general(xt, rot_ref[...], (((1,), (0,)), ((), ())),
                         preferred_element_type=jnp.float32)
    residual = xr
    qsum = jnp.zeros_like(xr)
    iota = lax.broadcasted_iota(jnp.int32, (_TB, _K), 1)
    idxs = []
    for i in range(_NUM_CB):
        cb = cb_ref[i]
        w2row = w2_ref[i]
        rw = lax.dot_general(residual, ncb_ref[i], (((1,), (1,)), ((), ())),
                             preferred_element_type=jnp.float32)
        rsq = jnp.sum(residual * residual, axis=1, keepdims=True)
        d = (rsq + w2row) + rw
        dmin = jnp.min(d, axis=1, keepdims=True)
        idx = jnp.min(jnp.where(d <= dmin, iota, _K), axis=1, keepdims=True)
        oh = (iota == idx).astype(jnp.float32)
        qi = lax.dot_general(oh, cb, (((1,), (0,)), ((), ())),
                             preferred_element_type=jnp.float32)
        residual = residual - qi
        qsum = qsum + qi
        idxs.append(idx)

    xraw = xraw_ref[...]
    out_ref[...] = xraw + (qsum - xraw)

    ent = jnp.zeros((_TB, 1), jnp.float32)
    for j in range(_NUM_CB):
        m = jnp.zeros((_TB, 1), jnp.float32)
        for k in range(_NUM_CB):
            m = m + (idxs[j] == idxs[k]).astype(jnp.float32)
        ent = ent - 0.25 * jnp.log(m * 0.25 + 1e-10)
    ent_sum = jnp.sum(ent)

    diff = qsum - xraw
    sq_sum = jnp.sum(diff * diff)

    lane = lax.broadcasted_iota(jnp.int32, (1, 128), 1)
    row = jnp.where(lane == 0, sq_sum, jnp.where(lane == 1, ent_sum, 0.0))
    part_ref[...] = row.reshape(1, 1, 128)


def kernel(x, rotation_matrix, codebooks):
    B, C, H, W = x.shape
    N = B * H * W
    xb = x.reshape(B, C, H * W)
    xraw = x.reshape(N, _D)
    w2 = jnp.sum(codebooks * codebooks, axis=-1)[:, None, :]
    ncb = codebooks * jnp.float32(-2.0)
    nblk = N // _TB
    q, part = pl.pallas_call(
        _vq_block,
        grid=(nblk,),
        in_specs=[
            pl.BlockSpec((1, _D, _TB), lambda i: (i, 0, 0)),
            pl.BlockSpec((_TB, _D), lambda i: (i, 0)),
            pl.BlockSpec((_D, _D), lambda i: (0, 0)),
            pl.BlockSpec((_NUM_CB, _K, _D), lambda i: (0, 0, 0)),
            pl.BlockSpec((_NUM_CB, _K, _D), lambda i: (0, 0, 0)),
            pl.BlockSpec((_NUM_CB, 1, _K), lambda i: (0, 0, 0)),
        ],
        out_specs=[
            pl.BlockSpec((_TB, _D), lambda i: (i, 0)),
            pl.BlockSpec((1, 1, 128), lambda i: (i, 0, 0)),
        ],
        out_shape=[
            jax.ShapeDtypeStruct((N, _D), jnp.float32),
            jax.ShapeDtypeStruct((nblk, 1, 128), jnp.float32),
        ],
        compiler_params=pltpu.CompilerParams(
            dimension_semantics=("arbitrary",)),
    )(xb, xraw, rotation_matrix, codebooks, ncb, w2)
    sq_sum = jnp.sum(part[:, 0, 0])
    ent_sum = jnp.sum(part[:, 0, 1])
    loss = (1.0 + _BETA) * sq_sum / jnp.float32(x.size)
    perplexity = jnp.exp(ent_sum)
    quant_out = q.reshape(x.shape)
    return loss, quant_out, perplexity


# speedup vs baseline: 1.1542x; 1.0047x over previous
import jax
import jax.numpy as jnp
from jax import lax
from jax.experimental import pallas as pl
from jax.experimental.pallas import tpu as pltpu

_NUM_CB = 4
_K = 1024
_D = 64
_TB = 1024
_BETA = 0.25


def _vq_block(xb_ref, xraw_ref, rot_ref, cb_ref, ncb_ref, w2_ref,
              out_ref, part_ref):
    xt = jnp.transpose(xb_ref[0], (1, 0))
    xr = lax.dot_general(xt, rot_ref[...], (((1,), (0,)), ((), ())),
                         preferred_element_type=jnp.float32)
    residual = xr
    qsum = jnp.zeros_like(xr)
    iota = lax.broadcasted_iota(jnp.int32, (_TB, _K), 1)
    idxs = []
    for i in range(_NUM_CB):
        cb = cb_ref[i]
        w2row = w2_ref[i]
        rw = lax.dot_general(residual, ncb_ref[i], (((1,), (1,)), ((), ())),
                             preferred_element_type=jnp.float32)
        rsq = jnp.sum(residual * residual, axis=1, keepdims=True)
        d = (rsq + w2row) + rw
        dmin = jnp.min(d, axis=1, keepdims=True)
        idx = jnp.min(jnp.where(d <= dmin, iota, _K), axis=1, keepdims=True)
        oh = (iota == idx).astype(jnp.float32)
        qi = lax.dot_general(oh, cb, (((1,), (0,)), ((), ())),
                             preferred_element_type=jnp.float32)
        residual = residual - qi
        qsum = qsum + qi
        idxs.append(idx)

    xraw = xraw_ref[...]
    out_ref[...] = xraw + (qsum - xraw)

    ent = jnp.zeros((_TB, 1), jnp.float32)
    for j in range(_NUM_CB):
        m = jnp.zeros((_TB, 1), jnp.float32)
        for k in range(_NUM_CB):
            m = m + (idxs[j] == idxs[k]).astype(jnp.float32)
        ent = ent - 0.25 * jnp.log(m * 0.25 + 1e-10)
    ent_sum = jnp.sum(ent)

    diff = qsum - xraw
    sq_sum = jnp.sum(diff * diff)

    lane = lax.broadcasted_iota(jnp.int32, (1, 128), 1)
    row = jnp.where(lane == 0, sq_sum, jnp.where(lane == 1, ent_sum, 0.0))
    part_ref[...] = row.reshape(1, 1, 128)


def kernel(x, rotation_matrix, codebooks):
    B, C, H, W = x.shape
    N = B * H * W
    xb = x.reshape(B, C, H * W)
    xraw = x.reshape(N, _D)
    w2 = jnp.sum(codebooks * codebooks, axis=-1)[:, None, :]
    ncb = codebooks * jnp.float32(-2.0)
    nblk = N // _TB
    q, part = pl.pallas_call(
        _vq_block,
        grid=(nblk,),
        in_specs=[
            pl.BlockSpec((1, _D, _TB), lambda i: (i, 0, 0)),
            pl.BlockSpec((_TB, _D), lambda i: (i, 0)),
            pl.BlockSpec((_D, _D), lambda i: (0, 0)),
            pl.BlockSpec((_NUM_CB, _K, _D), lambda i: (0, 0, 0)),
            pl.BlockSpec((_NUM_CB, _K, _D), lambda i: (0, 0, 0)),
            pl.BlockSpec((_NUM_CB, 1, _K), lambda i: (0, 0, 0)),
        ],
        out_specs=[
            pl.BlockSpec((_TB, _D), lambda i: (i, 0)),
            pl.BlockSpec((1, 1, 128), lambda i: (i, 0, 0)),
        ],
        out_shape=[
            jax.ShapeDtypeStruct((N, _D), jnp.float32),
            jax.ShapeDtypeStruct((nblk, 1, 128), jnp.float32),
        ],
        compiler_params=pltpu.CompilerParams(
            dimension_semantics=("parallel",)),
    )(xb, xraw, rotation_matrix, codebooks, ncb, w2)
    sq_sum = jnp.sum(part[:, 0, 0])
    ent_sum = jnp.sum(part[:, 0, 1])
    loss = (1.0 + _BETA) * sq_sum / jnp.float32(x.size)
    perplexity = jnp.exp(ent_sum)
    quant_out = q.reshape(x.shape)
    return loss, quant_out, perplexity


# speedup vs baseline: 1.3345x; 1.1562x over previous
import jax
import jax.numpy as jnp
from jax import lax
from jax.experimental import pallas as pl
from jax.experimental.pallas import tpu as pltpu

_NUM_CB = 4
_K = 1024
_D = 64
_TB = 2048
_BETA = 0.25


def _vq_block(xb_ref, xraw_ref, rot_ref, cb_ref, ncb_ref, w2_ref,
              out_ref, part_ref):
    xt = jnp.concatenate(
        [jnp.transpose(xb_ref[b], (1, 0)) for b in range(_TB // 1024)],
        axis=0)
    xr = lax.dot_general(xt, rot_ref[...], (((1,), (0,)), ((), ())),
                         preferred_element_type=jnp.float32)
    residual = xr
    qsum = jnp.zeros_like(xr)
    iota = lax.broadcasted_iota(jnp.int32, (_TB, _K), 1)
    idxs = []
    for i in range(_NUM_CB):
        cb = cb_ref[i]
        w2row = w2_ref[i]
        rw = lax.dot_general(residual, ncb_ref[i], (((1,), (1,)), ((), ())),
                             preferred_element_type=jnp.float32)
        rsq = jnp.sum(residual * residual, axis=1, keepdims=True)
        d = (rsq + w2row) + rw
        idx = jnp.argmin(d, axis=1).reshape(_TB, 1)
        oh = (iota == idx).astype(jnp.float32)
        qi = lax.dot_general(oh, cb, (((1,), (0,)), ((), ())),
                             preferred_element_type=jnp.float32)
        residual = residual - qi
        qsum = qsum + qi
        idxs.append(idx)

    xraw = xraw_ref[...]
    out_ref[...] = xraw + (qsum - xraw)

    ent = jnp.zeros((_TB, 1), jnp.float32)
    for j in range(_NUM_CB):
        m = jnp.zeros((_TB, 1), jnp.float32)
        for k in range(_NUM_CB):
            m = m + (idxs[j] == idxs[k]).astype(jnp.float32)
        ent = ent - 0.25 * jnp.log(m * 0.25 + 1e-10)
    ent_sum = jnp.sum(ent)

    diff = qsum - xraw
    sq_sum = jnp.sum(diff * diff)

    lane = lax.broadcasted_iota(jnp.int32, (1, 128), 1)
    row = jnp.where(lane == 0, sq_sum, jnp.where(lane == 1, ent_sum, 0.0))
    part_ref[...] = row.reshape(1, 1, 128)


def kernel(x, rotation_matrix, codebooks):
    B, C, H, W = x.shape
    N = B * H * W
    xb = x.reshape(B, C, H * W)
    xraw = x.reshape(N, _D)
    w2 = jnp.sum(codebooks * codebooks, axis=-1)[:, None, :]
    ncb = codebooks * jnp.float32(-2.0)
    nblk = N // _TB
    q, part = pl.pallas_call(
        _vq_block,
        grid=(nblk,),
        in_specs=[
            pl.BlockSpec((_TB // 1024, _D, 1024), lambda i: (i, 0, 0)),
            pl.BlockSpec((_TB, _D), lambda i: (i, 0)),
            pl.BlockSpec((_D, _D), lambda i: (0, 0)),
            pl.BlockSpec((_NUM_CB, _K, _D), lambda i: (0, 0, 0)),
            pl.BlockSpec((_NUM_CB, _K, _D), lambda i: (0, 0, 0)),
            pl.BlockSpec((_NUM_CB, 1, _K), lambda i: (0, 0, 0)),
        ],
        out_specs=[
            pl.BlockSpec((_TB, _D), lambda i: (i, 0)),
            pl.BlockSpec((1, 1, 128), lambda i: (i, 0, 0)),
        ],
        out_shape=[
            jax.ShapeDtypeStruct((N, _D), jnp.float32),
            jax.ShapeDtypeStruct((nblk, 1, 128), jnp.float32),
        ],
        compiler_params=pltpu.CompilerParams(
            dimension_semantics=("parallel",)),
    )(xb, xraw, rotation_matrix, codebooks, ncb, w2)
    sq_sum = jnp.sum(part[:, 0, 0])
    ent_sum = jnp.sum(part[:, 0, 1])
    loss = (1.0 + _BETA) * sq_sum / jnp.float32(x.size)
    perplexity = jnp.exp(ent_sum)
    quant_out = q.reshape(x.shape)
    return loss, quant_out, perplexity
